# Initial kernel scaffold; baseline (speedup 1.0000x reference)
import jax
import jax.numpy as jnp
from jax.experimental import pallas as pl


def kernel(input_ids, table, W1, b1, W2, b2):
    raise NotImplementedError("write your pallas kernel here")


# speedup vs baseline: 25.3087x; 25.3087x over previous
import functools

import jax
import jax.numpy as jnp
from jax.experimental import pallas as pl
from jax.experimental.pallas import tpu as pltpu
from jax.experimental.pallas import tpu_sc as plsc

_VOCAB = 1000000
_DIM = 128
_BATCH = 4096
_HIST = 200
_IDS = _BATCH * _HIST

_WINDOW = 128
_BB = 64


def _gather(table, flat_ids):
    mesh = plsc.VectorSubcoreMesh(core_axis_name="core",
                                  subcore_axis_name="subcore")

    @functools.partial(
        pl.kernel,
        out_type=jax.ShapeDtypeStruct((_IDS, _DIM), jnp.float32),
        mesh=mesh,
    )
    def sc_kernel(table_hbm, ids_hbm, out_hbm):
        def body(i_vmem, o_vmem):
            pltpu.sync_copy(table_hbm.at[i_vmem.at[0]], o_vmem)

        pltpu.emit_pipeline(
            body,
            grid=(_IDS // _WINDOW,),
            in_specs=[pl.BlockSpec((1, _WINDOW), index_map=lambda i: (0, i))],
            out_specs=[pl.BlockSpec((_WINDOW, _DIM),
                                    index_map=lambda i: (i, 0))],
            core_axis_name=("core", "subcore"),
            dimension_semantics=(pltpu.PARALLEL,),
        )(ids_hbm, out_hbm)

    return sc_kernel(table, flat_ids)


def _mlp_body(e_ref, w1t_ref, b1_ref, w2t_ref, b2_ref, s_ref, o_ref):
    h = jnp.dot(e_ref[...], w1t_ref[...],
                preferred_element_type=jnp.float32) + b1_ref[...]
    h = jnp.maximum(h, 0.0)
    hm = jnp.dot(s_ref[...], h, preferred_element_type=jnp.float32)
    o_ref[...] = jnp.dot(hm, w2t_ref[...],
                         preferred_element_type=jnp.float32) + b2_ref[...]


def _mlp(embeds, w1t, b1, w2t, b2, seg):
    grid = _BATCH // _BB
    return pl.pallas_call(
        _mlp_body,
        grid=(grid,),
        in_specs=[
            pl.BlockSpec((_BB * _HIST, _DIM), lambda i: (i, 0)),
            pl.BlockSpec((_DIM, _DIM), lambda i: (0, 0)),
            pl.BlockSpec((1, _DIM), lambda i: (0, 0)),
            pl.BlockSpec((_DIM, _DIM), lambda i: (0, 0)),
            pl.BlockSpec((1, _DIM), lambda i: (0, 0)),
            pl.BlockSpec((_BB, _BB * _HIST), lambda i: (0, 0)),
        ],
        out_specs=pl.BlockSpec((_BB, _DIM), lambda i: (i, 0)),
        out_shape=jax.ShapeDtypeStruct((_BATCH, _DIM), jnp.float32),
    )(embeds, w1t, b1, w2t, b2, seg)


def kernel(input_ids, table, W1, b1, W2, b2):
    flat_ids = input_ids.reshape(1, _IDS).astype(jnp.int32)
    col = jax.lax.broadcasted_iota(jnp.int32, (_BB, _BB * _HIST), 1)
    row = jax.lax.broadcasted_iota(jnp.int32, (_BB, _BB * _HIST), 0)
    seg = jnp.where(col // _HIST == row, jnp.float32(1.0 / _HIST),
                    jnp.float32(0.0))
    embeds = _gather(table, flat_ids)
    return _mlp(embeds, W1.T, b1.reshape(1, _DIM), W2.T,
                b2.reshape(1, _DIM), seg)


# speedup vs baseline: 26.9667x; 1.0655x over previous
import functools

import jax
import jax.numpy as jnp
from jax.experimental import pallas as pl
from jax.experimental.pallas import tpu as pltpu
from jax.experimental.pallas import tpu_sc as plsc

_VOCAB = 1000000
_DIM = 128
_BATCH = 4096
_HIST = 200
_IDS = _BATCH * _HIST

_WINDOW = 128
_BB = 64
_NCHUNK = 4
_CB = _BATCH // _NCHUNK
_CIDS = _CB * _HIST


def _gather(table, flat_ids):
    mesh = plsc.VectorSubcoreMesh(core_axis_name="core",
                                  subcore_axis_name="subcore")

    @functools.partial(
        pl.kernel,
        out_type=jax.ShapeDtypeStruct((_CIDS, _DIM), jnp.float32),
        mesh=mesh,
    )
    def sc_kernel(table_hbm, ids_hbm, out_hbm):
        def body(i_vmem, o_vmem):
            pltpu.sync_copy(table_hbm.at[i_vmem.at[0]], o_vmem)

        pltpu.emit_pipeline(
            body,
            grid=(_CIDS // _WINDOW,),
            in_specs=[pl.BlockSpec((1, _WINDOW), index_map=lambda i: (0, i))],
            out_specs=[pl.BlockSpec((_WINDOW, _DIM),
                                    index_map=lambda i: (i, 0))],
            core_axis_name=("core", "subcore"),
            dimension_semantics=(pltpu.PARALLEL,),
        )(ids_hbm, out_hbm)

    return sc_kernel(table, flat_ids)


def _mlp_body(e_ref, w1t_ref, b1_ref, w2t_ref, b2_ref, s_ref, o_ref):
    h = jnp.dot(e_ref[...], w1t_ref[...],
                preferred_element_type=jnp.float32) + b1_ref[...]
    h = jnp.maximum(h, 0.0)
    hm = jnp.dot(s_ref[...], h, preferred_element_type=jnp.float32)
    o_ref[...] = jnp.dot(hm, w2t_ref[...],
                         preferred_element_type=jnp.float32) + b2_ref[...]


def _mlp(embeds, w1t, b1, w2t, b2, seg):
    grid = _CB // _BB
    return pl.pallas_call(
        _mlp_body,
        grid=(grid,),
        in_specs=[
            pl.BlockSpec((_BB * _HIST, _DIM), lambda i: (i, 0)),
            pl.BlockSpec((_DIM, _DIM), lambda i: (0, 0)),
            pl.BlockSpec((1, _DIM), lambda i: (0, 0)),
            pl.BlockSpec((_DIM, _DIM), lambda i: (0, 0)),
            pl.BlockSpec((1, _DIM), lambda i: (0, 0)),
            pl.BlockSpec((_BB, _BB * _HIST), lambda i: (0, 0)),
        ],
        out_specs=pl.BlockSpec((_BB, _DIM), lambda i: (i, 0)),
        out_shape=jax.ShapeDtypeStruct((_CB, _DIM), jnp.float32),
    )(embeds, w1t, b1, w2t, b2, seg)


def kernel(input_ids, table, W1, b1, W2, b2):
    flat_ids = input_ids.reshape(_NCHUNK, 1, _CIDS).astype(jnp.int32)
    col = jax.lax.broadcasted_iota(jnp.int32, (_BB, _BB * _HIST), 1)
    row = jax.lax.broadcasted_iota(jnp.int32, (_BB, _BB * _HIST), 0)
    seg = jnp.where(col // _HIST == row, jnp.float32(1.0 / _HIST),
                    jnp.float32(0.0))
    w1t, w2t = W1.T, W2.T
    b1r, b2r = b1.reshape(1, _DIM), b2.reshape(1, _DIM)
    outs = []
    for c in range(_NCHUNK):
        emb_c = _gather(table, flat_ids[c])
        outs.append(_mlp(emb_c, w1t, b1r, w2t, b2r, seg))
    return jnp.concatenate(outs, axis=0)
